# Initial kernel scaffold; baseline (speedup 1.0000x reference)
import jax
import jax.numpy as jnp
from jax.experimental import pallas as pl


def kernel(x, y, neg_samples, W_word, W_ctx):
    raise NotImplementedError("write your pallas kernel here")


# speedup vs baseline: 1.9141x; 1.9141x over previous
import functools

import jax
import jax.numpy as jnp
from jax import lax
from jax.experimental import pallas as pl
from jax.experimental.pallas import tpu as pltpu
from jax.experimental.pallas import tpu_sc as plsc

_B = 4096
_E = 128
_S = _B // _E
_NEG = 20
_NC, _NS = 2, 16
_NW = _NC * _NS
_W_ROWS = 2 * _B
_W_PER = _W_ROWS // _NW
_C_ROWS = _B + 256
_C_PER = _C_ROWS // _NW

def _sc_gather_body(ww_hbm, wc_hbm, idxw_hbm, idxc_hbm, outw_hbm, outc_hbm,
                    idxw_v, roww_v, idxc_v, rowc_v, sem_w, sem_c):
    wid = lax.axis_index("s") * _NC + lax.axis_index("c")
    bw = wid * _W_PER
    bc = wid * _C_PER
    pltpu.sync_copy(idxw_hbm.at[pl.ds(bw, _W_PER)], idxw_v)
    pltpu.sync_copy(idxc_hbm.at[pl.ds(bc, _C_PER)], idxc_v)
    copies = []
    for j in range(_W_PER // 128):
        copies.append(pltpu.async_copy(
            ww_hbm.at[idxw_v.at[pl.ds(j * 128, 128)]],
            roww_v.at[pl.ds(j * 128, 128)], sem_w))
    copies.append(pltpu.async_copy(
        wc_hbm.at[idxc_v.at[pl.ds(0, 128)]], rowc_v.at[pl.ds(0, 128)], sem_c))
    copies.append(pltpu.async_copy(
        wc_hbm.at[idxc_v.at[pl.ds(128, _C_PER - 128)]],
        rowc_v.at[pl.ds(128, _C_PER - 128)], sem_c))
    for cp in copies:
        cp.wait()
    pltpu.sync_copy(roww_v, outw_hbm.at[pl.ds(bw, _W_PER)])
    pltpu.sync_copy(rowc_v, outc_hbm.at[pl.ds(bc, _C_PER)])


@functools.cache
def _sc_gather():
    mesh = plsc.VectorSubcoreMesh(core_axis_name="c", subcore_axis_name="s",
                                  num_cores=_NC, num_subcores=_NS)
    return pl.kernel(
        _sc_gather_body,
        out_type=(
            jax.ShapeDtypeStruct((_W_ROWS, _E), jnp.float32),
            jax.ShapeDtypeStruct((_C_ROWS, _E), jnp.float32),
        ),
        mesh=mesh,
        scratch_types=[
            pltpu.VMEM((_W_PER,), jnp.int32),
            pltpu.VMEM((_W_PER, _E), jnp.float32),
            pltpu.VMEM((_C_PER,), jnp.int32),
            pltpu.VMEM((_C_PER, _E), jnp.float32),
            pltpu.SemaphoreType.DMA,
            pltpu.SemaphoreType.DMA,
        ],
    )


def _nls(z):
    return jnp.maximum(-z, 0.0) + jnp.log(1.0 + jnp.exp(-jnp.abs(z)))


def _tc_body(xn_ref, xp_ref, yg_ref, ng_ref, out_ref):
    b = pl.program_id(0)
    xn = xn_ref[...]
    xp = xp_ref[...]
    yg = yg_ref[...]
    ng = ng_ref[...]
    m = jnp.dot(xn, yg, preferred_element_type=jnp.float32)
    eye = (lax.broadcasted_iota(jnp.int32, (_E, _E), 0)
           == lax.broadcasted_iota(jnp.int32, (_E, _E), 1))
    diag = jnp.sum(jnp.where(eye, m, 0.0), axis=0, keepdims=True)
    pos = jnp.sum(_nls(diag)) * (1.0 / _B)
    nb = jnp.dot(ng, xp, preferred_element_type=jnp.float32)
    nmask = lax.broadcasted_iota(jnp.int32, (_S, _E), 0) < _NEG
    neg = jnp.sum(jnp.where(nmask, _nls(-nb), 0.0))

    @pl.when(b == 0)
    def _init():
        out_ref[0, 0] = 0.0

    out_ref[0, 0] += pos + neg


def _tc_reduce(xn, xp, yg, ng):
    return pl.pallas_call(
        _tc_body,
        grid=(_S,),
        in_specs=[
            pl.BlockSpec((_E, _E), lambda b: (b, 0)),
            pl.BlockSpec((_E, _E), lambda b: (b, 0)),
            pl.BlockSpec((_E, _E), lambda b: (b, 0)),
            pl.BlockSpec((_S, _E), lambda b: (0, 0)),
        ],
        out_specs=pl.BlockSpec((1, 1), lambda b: (0, 0),
                               memory_space=pltpu.SMEM),
        out_shape=jax.ShapeDtypeStruct((1, 1), jnp.float32),
    )(xn, xp, yg, ng)


def kernel(x, y, neg_samples, W_word, W_ctx):
    x = x.astype(jnp.int32)
    y = y.astype(jnp.int32)
    neg = neg_samples.astype(jnp.int32)
    x_perm = x.reshape(_E, _S).T.reshape(-1)
    y_perm = y.reshape(_E, _S).T.reshape(-1)
    idx_w = jnp.concatenate([x, x_perm])
    idx_c = jnp.concatenate(
        [y_perm, neg, jnp.zeros((256 - _NEG,), jnp.int32)])
    outw, outc = _sc_gather()(W_word, W_ctx, idx_w, idx_c)
    xn = outw[:_B]
    xp = outw[_B:]
    yg = outc[:_B]
    ng = outc[_B:_B + _S]
    res = _tc_reduce(xn, xp, yg, ng)
    return res[0, 0]


# speedup vs baseline: 2.1792x; 1.1385x over previous
import functools

import jax
import jax.numpy as jnp
from jax import lax
from jax.experimental import pallas as pl
from jax.experimental.pallas import tpu as pltpu
from jax.experimental.pallas import tpu_sc as plsc

_B = 4096
_E = 128
_S = _B // _E
_NEG = 20
_NC, _NS = 2, 16
_NW = _NC * _NS
_W_PER = _B // _NW
_C_ROWS = _B + 256
_C_PER = _C_ROWS // _NW


def _sc_gather_body(ww_hbm, wc_hbm, idxw_hbm, idxc_hbm, outw_hbm, outc_hbm,
                    idxw_v, roww_v, idxc_v, rowc_v,
                    sem_w, sem_c, sem_o):
    wid = lax.axis_index("s") * _NC + lax.axis_index("c")
    bw = wid * _W_PER
    bc = wid * _C_PER
    pltpu.sync_copy(idxw_hbm.at[pl.ds(bw, _W_PER)], idxw_v)
    pltpu.sync_copy(idxc_hbm.at[pl.ds(bc, _C_PER)], idxc_v)
    cp_w = pltpu.async_copy(ww_hbm.at[idxw_v], roww_v, sem_w)
    cp_c0 = pltpu.async_copy(
        wc_hbm.at[idxc_v.at[pl.ds(0, 128)]], rowc_v.at[pl.ds(0, 128)], sem_c)
    cp_c1 = pltpu.async_copy(
        wc_hbm.at[idxc_v.at[pl.ds(128, _C_PER - 128)]],
        rowc_v.at[pl.ds(128, _C_PER - 128)], sem_c)
    cp_w.wait()
    wb_w = pltpu.async_copy(roww_v, outw_hbm.at[pl.ds(bw, _W_PER)], sem_o)
    cp_c0.wait()
    cp_c1.wait()
    wb_c = pltpu.async_copy(rowc_v, outc_hbm.at[pl.ds(bc, _C_PER)], sem_o)
    wb_w.wait()
    wb_c.wait()


@functools.cache
def _sc_gather():
    mesh = plsc.VectorSubcoreMesh(core_axis_name="c", subcore_axis_name="s",
                                  num_cores=_NC, num_subcores=_NS)
    return pl.kernel(
        _sc_gather_body,
        out_type=(
            jax.ShapeDtypeStruct((_B, _E), jnp.float32),
            jax.ShapeDtypeStruct((_C_ROWS, _E), jnp.float32),
        ),
        mesh=mesh,
        scratch_types=[
            pltpu.VMEM((_W_PER,), jnp.int32),
            pltpu.VMEM((_W_PER, _E), jnp.float32),
            pltpu.VMEM((_C_PER,), jnp.int32),
            pltpu.VMEM((_C_PER, _E), jnp.float32),
            pltpu.SemaphoreType.DMA,
            pltpu.SemaphoreType.DMA,
            pltpu.SemaphoreType.DMA,
        ],
    )


def _nls(z):
    return jnp.maximum(-z, 0.0) + jnp.log(1.0 + jnp.exp(-jnp.abs(z)))


def _tc_body(xn_ref, xp_ref, yg_ref, ng_ref, out_ref):
    b = pl.program_id(0)
    xn = xn_ref[...]
    xp = xp_ref[...]
    yg = yg_ref[...]
    ng = ng_ref[...]
    m = jnp.dot(xn, yg, preferred_element_type=jnp.float32)
    eye = (lax.broadcasted_iota(jnp.int32, (_E, _E), 0)
           == lax.broadcasted_iota(jnp.int32, (_E, _E), 1))
    diag = jnp.sum(jnp.where(eye, m, 0.0), axis=0, keepdims=True)
    pos = jnp.sum(_nls(diag)) * (1.0 / _B)
    nb = jnp.dot(ng, xp, preferred_element_type=jnp.float32)
    nmask = lax.broadcasted_iota(jnp.int32, (_S, _E), 0) < _NEG
    neg = jnp.sum(jnp.where(nmask, _nls(-nb), 0.0))

    @pl.when(b == 0)
    def _init():
        out_ref[0, 0] = 0.0

    out_ref[0, 0] += pos + neg


def _tc_reduce(outw, outw3, outc):
    return pl.pallas_call(
        _tc_body,
        grid=(_S,),
        in_specs=[
            pl.BlockSpec((_E, _E), lambda b: (b, 0)),
            pl.BlockSpec((_E, _E), lambda b: (0, b)),
            pl.BlockSpec((_E, _E), lambda b: (b, 0)),
            pl.BlockSpec((_S, _E), lambda b: (_B // _S, 0)),
        ],
        out_specs=pl.BlockSpec((1, 1), lambda b: (0, 0),
                               memory_space=pltpu.SMEM),
        out_shape=jax.ShapeDtypeStruct((1, 1), jnp.float32),
    )(outw, outw3, outc, outc)


def kernel(x, y, neg_samples, W_word, W_ctx):
    x = x.astype(jnp.int32)
    y = y.astype(jnp.int32)
    neg = neg_samples.astype(jnp.int32)
    y_perm = y.reshape(_E, _S).T.reshape(-1)
    idx_c = jnp.concatenate(
        [y_perm, neg, jnp.zeros((256 - _NEG,), jnp.int32)])
    outw, outc = _sc_gather()(W_word, W_ctx, x, idx_c)
    res = _tc_reduce(outw, outw.reshape(_E, _B), outc)
    return res[0, 0]


# speedup vs baseline: 3.9537x; 1.8143x over previous
import functools

import jax
import jax.numpy as jnp
from jax import lax
from jax.experimental import pallas as pl
from jax.experimental.pallas import tpu as pltpu
from jax.experimental.pallas import tpu_sc as plsc

_B = 4096
_E = 128
_S = _B // _E
_NEG = 20
_NC, _NS = 2, 16
_NW = _NC * _NS
_PER = _B // _NW
_TCG = 4


def _sc_gather_body(ww_hbm, wc_hbm, x_hbm, y_hbm, neg_hbm,
                    outx_hbm, outxp_hbm, outyg_hbm, outn_hbm,
                    idxx_v, idxy_v, idxn_v, didx_v, gx_v, gy_v, gn_v,
                    sem_x, sem_y, sem_n, sem_o):
    wid = lax.axis_index("s") * _NC + lax.axis_index("c")
    base = wid * _PER
    pltpu.sync_copy(x_hbm.at[pl.ds(base, _PER)], idxx_v)
    pltpu.sync_copy(y_hbm.at[pl.ds(base, _PER)], idxy_v)
    cp_x = pltpu.async_copy(ww_hbm.at[idxx_v], gx_v, sem_x)
    cp_y = pltpu.async_copy(wc_hbm.at[idxy_v], gy_v, sem_y)

    lane = lax.broadcasted_iota(jnp.int32, (16,), 0) * 128
    for t in range(_PER // 16):
        didx_v[pl.ds(t * 16, 16)] = lane + (2048 * (t % 2) + 4 * wid + t // 2)

    @pl.when(wid == 0)
    def _neg_path():
        pltpu.sync_copy(neg_hbm, idxn_v)
        pltpu.async_copy(wc_hbm.at[idxn_v], gn_v, sem_n).wait()
        pltpu.sync_copy(gn_v, outn_hbm)

    cp_x.wait()
    wb_xn = pltpu.async_copy(gx_v, outx_hbm.at[pl.ds(base, _PER)], sem_o)
    wb_xp = pltpu.async_copy(gx_v, outxp_hbm.at[didx_v], sem_o)
    cp_y.wait()
    wb_yg = pltpu.async_copy(gy_v, outyg_hbm.at[didx_v], sem_o)
    wb_xn.wait()
    wb_xp.wait()
    wb_yg.wait()


@functools.cache
def _sc_gather():
    mesh = plsc.VectorSubcoreMesh(core_axis_name="c", subcore_axis_name="s",
                                  num_cores=_NC, num_subcores=_NS)
    return pl.kernel(
        _sc_gather_body,
        out_type=(
            jax.ShapeDtypeStruct((_B, _E), jnp.float32),
            jax.ShapeDtypeStruct((_B, _E), jnp.float32),
            jax.ShapeDtypeStruct((_B, _E), jnp.float32),
            jax.ShapeDtypeStruct((_NEG, _E), jnp.float32),
        ),
        mesh=mesh,
        scratch_types=[
            pltpu.VMEM((_PER,), jnp.int32),
            pltpu.VMEM((_PER,), jnp.int32),
            pltpu.VMEM((_NEG,), jnp.int32),
            pltpu.VMEM((_PER,), jnp.int32),
            pltpu.VMEM((_PER, _E), jnp.float32),
            pltpu.VMEM((_PER, _E), jnp.float32),
            pltpu.VMEM((_NEG, _E), jnp.float32),
            pltpu.SemaphoreType.DMA,
            pltpu.SemaphoreType.DMA,
            pltpu.SemaphoreType.DMA,
            pltpu.SemaphoreType.DMA,
        ],
    )


def _nls(z):
    return jnp.maximum(-z, 0.0) + jnp.log(1.0 + jnp.exp(-jnp.abs(z)))


def _tc_body(xn_ref, xp_ref, yg_ref, ng_ref, out_ref):
    g = pl.program_id(0)
    ng = ng_ref[...]
    eye = (lax.broadcasted_iota(jnp.int32, (_E, _E), 0)
           == lax.broadcasted_iota(jnp.int32, (_E, _E), 1))
    acc = jnp.float32(0.0)
    for j in range(_TCG):
        sl = pl.ds(j * _E, _E)
        xn = xn_ref[sl, :]
        xp = xp_ref[sl, :]
        yg = yg_ref[sl, :]
        m = jnp.dot(xn, yg, preferred_element_type=jnp.float32)
        diag = jnp.sum(jnp.where(eye, m, 0.0), axis=0, keepdims=True)
        nb = jnp.dot(ng, xp, preferred_element_type=jnp.float32)
        acc += jnp.sum(_nls(diag)) * (1.0 / _B) + jnp.sum(_nls(-nb))

    @pl.when(g == 0)
    def _init():
        out_ref[0, 0] = 0.0

    out_ref[0, 0] += acc


def _tc_reduce(outx, outxp, outyg, outn):
    blk = _TCG * _E
    return pl.pallas_call(
        _tc_body,
        grid=(_S // _TCG,),
        in_specs=[
            pl.BlockSpec((blk, _E), lambda g: (g, 0)),
            pl.BlockSpec((blk, _E), lambda g: (g, 0)),
            pl.BlockSpec((blk, _E), lambda g: (g, 0)),
            pl.BlockSpec((_NEG, _E), lambda g: (0, 0)),
        ],
        out_specs=pl.BlockSpec((1, 1), lambda g: (0, 0),
                               memory_space=pltpu.SMEM),
        out_shape=jax.ShapeDtypeStruct((1, 1), jnp.float32),
    )(outx, outxp, outyg, outn)


def kernel(x, y, neg_samples, W_word, W_ctx):
    x = x.astype(jnp.int32)
    y = y.astype(jnp.int32)
    neg = neg_samples.astype(jnp.int32)
    outx, outxp, outyg, outn = _sc_gather()(W_word, W_ctx, x, y, neg)
    res = _tc_reduce(outx, outxp, outyg, outn)
    return res[0, 0]


# speedup vs baseline: 4.2507x; 1.0751x over previous
import functools

import jax
import jax.numpy as jnp
from jax import lax
from jax.experimental import pallas as pl
from jax.experimental.pallas import tpu as pltpu
from jax.experimental.pallas import tpu_sc as plsc

_B = 4096
_E = 128
_S = _B // _E
_NEG = 20
_NC, _NS = 2, 16
_NW = _NC * _NS
_PER = _B // _NW
_TCG = 8


def _sc_gather_body(ww_hbm, wc_hbm, x_hbm, y_hbm, neg_hbm,
                    outx_hbm, outxp_hbm, outyg_hbm, outn_hbm,
                    idxx_v, idxy_v, idxn_v, didx_v, gx_v, gy_v, gn_v,
                    sem_x, sem_y, sem_n, sem_o):
    wid = lax.axis_index("s") * _NC + lax.axis_index("c")
    base = wid * _PER
    pltpu.sync_copy(x_hbm.at[pl.ds(base, _PER)], idxx_v)
    pltpu.sync_copy(y_hbm.at[pl.ds(base, _PER)], idxy_v)
    cp_x = pltpu.async_copy(ww_hbm.at[idxx_v], gx_v, sem_x)
    cp_y = pltpu.async_copy(wc_hbm.at[idxy_v], gy_v, sem_y)

    lane = lax.broadcasted_iota(jnp.int32, (16,), 0) * 128
    for t in range(_PER // 16):
        didx_v[pl.ds(t * 16, 16)] = lane + (2048 * (t % 2) + 4 * wid + t // 2)

    @pl.when(wid == 0)
    def _neg_path():
        pltpu.sync_copy(neg_hbm, idxn_v)
        pltpu.async_copy(wc_hbm.at[idxn_v], gn_v, sem_n).wait()
        pltpu.sync_copy(gn_v, outn_hbm)

    cp_x.wait()
    wb_xn = pltpu.async_copy(gx_v, outx_hbm.at[pl.ds(base, _PER)], sem_o)
    wb_xp = pltpu.async_copy(gx_v, outxp_hbm.at[didx_v], sem_o)
    cp_y.wait()
    wb_yg = pltpu.async_copy(gy_v, outyg_hbm.at[didx_v], sem_o)
    wb_xn.wait()
    wb_xp.wait()
    wb_yg.wait()


@functools.cache
def _sc_gather():
    mesh = plsc.VectorSubcoreMesh(core_axis_name="c", subcore_axis_name="s",
                                  num_cores=_NC, num_subcores=_NS)
    return pl.kernel(
        _sc_gather_body,
        out_type=(
            jax.ShapeDtypeStruct((_B, _E), jnp.float32),
            jax.ShapeDtypeStruct((_B, _E), jnp.float32),
            jax.ShapeDtypeStruct((_B, _E), jnp.float32),
            jax.ShapeDtypeStruct((_NEG, _E), jnp.float32),
        ),
        mesh=mesh,
        scratch_types=[
            pltpu.VMEM((_PER,), jnp.int32),
            pltpu.VMEM((_PER,), jnp.int32),
            pltpu.VMEM((_NEG,), jnp.int32),
            pltpu.VMEM((_PER,), jnp.int32),
            pltpu.VMEM((_PER, _E), jnp.float32),
            pltpu.VMEM((_PER, _E), jnp.float32),
            pltpu.VMEM((_NEG, _E), jnp.float32),
            pltpu.SemaphoreType.DMA,
            pltpu.SemaphoreType.DMA,
            pltpu.SemaphoreType.DMA,
            pltpu.SemaphoreType.DMA,
        ],
    )


def _nls(z):
    return jnp.maximum(-z, 0.0) + jnp.log(1.0 + jnp.exp(-jnp.abs(z)))


def _tc_body(xn_ref, xp_ref, yg_ref, ng_ref, out_ref):
    g = pl.program_id(0)
    ng = ng_ref[...]
    eye = (lax.broadcasted_iota(jnp.int32, (_E, _E), 0)
           == lax.broadcasted_iota(jnp.int32, (_E, _E), 1))
    acc = jnp.float32(0.0)
    for j in range(_TCG):
        sl = pl.ds(j * _E, _E)
        xn = xn_ref[sl, :]
        xp = xp_ref[sl, :]
        yg = yg_ref[sl, :]
        m = jnp.dot(xn, yg, preferred_element_type=jnp.float32)
        diag = jnp.sum(jnp.where(eye, m, 0.0), axis=0, keepdims=True)
        nb = jnp.dot(ng, xp, preferred_element_type=jnp.float32)
        acc += jnp.sum(_nls(diag)) * (1.0 / _B) + jnp.sum(_nls(-nb))

    @pl.when(g == 0)
    def _init():
        out_ref[0, 0] = 0.0

    out_ref[0, 0] += acc


def _tc_reduce(outx, outxp, outyg, outn):
    blk = _TCG * _E
    return pl.pallas_call(
        _tc_body,
        grid=(_S // _TCG,),
        in_specs=[
            pl.BlockSpec((blk, _E), lambda g: (g, 0)),
            pl.BlockSpec((blk, _E), lambda g: (g, 0)),
            pl.BlockSpec((blk, _E), lambda g: (g, 0)),
            pl.BlockSpec((_NEG, _E), lambda g: (0, 0)),
        ],
        out_specs=pl.BlockSpec((1, 1), lambda g: (0, 0),
                               memory_space=pltpu.SMEM),
        out_shape=jax.ShapeDtypeStruct((1, 1), jnp.float32),
    )(outx, outxp, outyg, outn)


def kernel(x, y, neg_samples, W_word, W_ctx):
    x = x.astype(jnp.int32)
    y = y.astype(jnp.int32)
    neg = neg_samples.astype(jnp.int32)
    outx, outxp, outyg, outn = _sc_gather()(W_word, W_ctx, x, y, neg)
    res = _tc_reduce(outx, outxp, outyg, outn)
    return res[0, 0]


# speedup vs baseline: 4.3577x; 1.0252x over previous
import functools

import jax
import jax.numpy as jnp
from jax import lax
from jax.experimental import pallas as pl
from jax.experimental.pallas import tpu as pltpu
from jax.experimental.pallas import tpu_sc as plsc

_B = 4096
_E = 128
_S = _B // _E
_NEG = 20
_NC, _NS = 2, 16
_NW = _NC * _NS
_PER = _B // _NW
_TCG = 16


def _sc_gather_body(ww_hbm, wc_hbm, x_hbm, y_hbm, neg_hbm,
                    outx_hbm, outxp_hbm, outyg_hbm, outn_hbm,
                    idxx_v, idxy_v, idxn_v, didx_v, gx_v, gy_v, gn_v,
                    sem_x, sem_y, sem_n, sem_o):
    wid = lax.axis_index("s") * _NC + lax.axis_index("c")
    base = wid * _PER
    pltpu.sync_copy(x_hbm.at[pl.ds(base, _PER)], idxx_v)
    pltpu.sync_copy(y_hbm.at[pl.ds(base, _PER)], idxy_v)
    cp_x = pltpu.async_copy(ww_hbm.at[idxx_v], gx_v, sem_x)
    cp_y = pltpu.async_copy(wc_hbm.at[idxy_v], gy_v, sem_y)

    lane = lax.broadcasted_iota(jnp.int32, (16,), 0) * 128
    for t in range(_PER // 16):
        didx_v[pl.ds(t * 16, 16)] = lane + (2048 * (t % 2) + 4 * wid + t // 2)

    @pl.when(wid == 0)
    def _neg_path():
        pltpu.sync_copy(neg_hbm, idxn_v)
        pltpu.async_copy(wc_hbm.at[idxn_v], gn_v, sem_n).wait()
        pltpu.sync_copy(gn_v, outn_hbm)

    cp_x.wait()
    wb_xn = pltpu.async_copy(gx_v, outx_hbm.at[pl.ds(base, _PER)], sem_o)
    wb_xp = pltpu.async_copy(gx_v, outxp_hbm.at[didx_v], sem_o)
    cp_y.wait()
    wb_yg = pltpu.async_copy(gy_v, outyg_hbm.at[didx_v], sem_o)
    wb_xn.wait()
    wb_xp.wait()
    wb_yg.wait()


@functools.cache
def _sc_gather():
    mesh = plsc.VectorSubcoreMesh(core_axis_name="c", subcore_axis_name="s",
                                  num_cores=_NC, num_subcores=_NS)
    return pl.kernel(
        _sc_gather_body,
        out_type=(
            jax.ShapeDtypeStruct((_B, _E), jnp.float32),
            jax.ShapeDtypeStruct((_B, _E), jnp.float32),
            jax.ShapeDtypeStruct((_B, _E), jnp.float32),
            jax.ShapeDtypeStruct((_NEG, _E), jnp.float32),
        ),
        mesh=mesh,
        scratch_types=[
            pltpu.VMEM((_PER,), jnp.int32),
            pltpu.VMEM((_PER,), jnp.int32),
            pltpu.VMEM((_NEG,), jnp.int32),
            pltpu.VMEM((_PER,), jnp.int32),
            pltpu.VMEM((_PER, _E), jnp.float32),
            pltpu.VMEM((_PER, _E), jnp.float32),
            pltpu.VMEM((_NEG, _E), jnp.float32),
            pltpu.SemaphoreType.DMA,
            pltpu.SemaphoreType.DMA,
            pltpu.SemaphoreType.DMA,
            pltpu.SemaphoreType.DMA,
        ],
    )


def _nls(z):
    return jnp.maximum(-z, 0.0) + jnp.log(1.0 + jnp.exp(-jnp.abs(z)))


def _tc_body(xn_ref, xp_ref, yg_ref, ng_ref, out_ref):
    g = pl.program_id(0)
    ng = ng_ref[...]
    eye = (lax.broadcasted_iota(jnp.int32, (_E, _E), 0)
           == lax.broadcasted_iota(jnp.int32, (_E, _E), 1))
    acc = jnp.float32(0.0)
    for j in range(_TCG):
        sl = pl.ds(j * _E, _E)
        xn = xn_ref[sl, :]
        xp = xp_ref[sl, :]
        yg = yg_ref[sl, :]
        m = jnp.dot(xn, yg, preferred_element_type=jnp.float32)
        diag = jnp.sum(jnp.where(eye, m, 0.0), axis=0, keepdims=True)
        nb = jnp.dot(ng, xp, preferred_element_type=jnp.float32)
        acc += jnp.sum(_nls(diag)) * (1.0 / _B) + jnp.sum(_nls(-nb))

    @pl.when(g == 0)
    def _init():
        out_ref[0, 0] = 0.0

    out_ref[0, 0] += acc


def _tc_reduce(outx, outxp, outyg, outn):
    blk = _TCG * _E
    return pl.pallas_call(
        _tc_body,
        grid=(_S // _TCG,),
        in_specs=[
            pl.BlockSpec((blk, _E), lambda g: (g, 0)),
            pl.BlockSpec((blk, _E), lambda g: (g, 0)),
            pl.BlockSpec((blk, _E), lambda g: (g, 0)),
            pl.BlockSpec((_NEG, _E), lambda g: (0, 0)),
        ],
        out_specs=pl.BlockSpec((1, 1), lambda g: (0, 0),
                               memory_space=pltpu.SMEM),
        out_shape=jax.ShapeDtypeStruct((1, 1), jnp.float32),
    )(outx, outxp, outyg, outn)


def kernel(x, y, neg_samples, W_word, W_ctx):
    x = x.astype(jnp.int32)
    y = y.astype(jnp.int32)
    neg = neg_samples.astype(jnp.int32)
    outx, outxp, outyg, outn = _sc_gather()(W_word, W_ctx, x, y, neg)
    res = _tc_reduce(outx, outxp, outyg, outn)
    return res[0, 0]
